# baseline (speedup 1.0000x reference)
import functools

import jax
import jax.numpy as jnp
from jax import lax
from jax.experimental import pallas as pl
from jax.experimental.pallas import tpu as pltpu
from jax.experimental.pallas import tpu_sc as plsc

VOCAB = 100000
EMBD = 128
CTX = 10
HID = 512
NIDX = 2 * CTX

NQ = 16
NST = 4
NBUF = 3

SC_C0 = 67200
SC_CW = 1024
SC_NW = 32
SC_COLS = SC_CW * SC_NW
C_TC = SC_C0
TAIL = VOCAB - SC_C0 - SC_COLS

SC_KT = 16
SC_RING = 3
SC_NCH = HID // SC_KT


def _sc_gather(table, idx):
    mesh = plsc.VectorSubcoreMesh(core_axis_name="c", subcore_axis_name="s")

    @functools.partial(
        pl.kernel,
        mesh=mesh,
        out_type=jax.ShapeDtypeStruct((NIDX, EMBD), jnp.float32),
        scratch_types=[
            pltpu.VMEM((NIDX,), jnp.int32),
            pltpu.VMEM((NIDX, EMBD), jnp.float32),
            pltpu.SemaphoreType.DMA,
        ],
    )
    def gather_k(table_hbm, idx_hbm, out_hbm, idx_v, rows_v, sem):
        wid = lax.axis_index("s") * 2 + lax.axis_index("c")

        @pl.when(wid == 0)
        def _():
            pltpu.sync_copy(idx_hbm, idx_v)
            pltpu.async_copy(table_hbm.at[idx_v], rows_v, sem).wait()
            pltpu.sync_copy(rows_v, out_hbm)

    return gather_k(table, idx)


def _sc_matvec(W2, hid_b):
    mesh = plsc.VectorSubcoreMesh(core_axis_name="c", subcore_axis_name="s")

    @functools.partial(
        pl.kernel,
        mesh=mesh,
        out_type=jax.ShapeDtypeStruct((SC_COLS,), jnp.float32),
        scratch_types=[
            pltpu.VMEM((SC_RING, SC_KT, SC_CW), jnp.float32),
            pltpu.VMEM((HID, 16), jnp.float32),
            pltpu.VMEM((SC_CW,), jnp.float32),
            pltpu.SemaphoreType.DMA((SC_RING,)),
        ],
    )
    def k(w2_hbm, hid_hbm, out_hbm, wbuf, hbuf, acc, sems):
        wid = lax.axis_index("s") * 2 + lax.axis_index("c")
        c0 = SC_C0 + wid * SC_CW

        def mk(i, slot):
            return pltpu.make_async_copy(
                w2_hbm.at[pl.ds(i * SC_KT, SC_KT), pl.ds(c0, SC_CW)],
                wbuf.at[slot],
                sems.at[slot],
            )

        pltpu.sync_copy(hid_hbm, hbuf)
        for i in range(SC_RING - 1):
            mk(i, i).start()

        z = jnp.zeros((16,), jnp.float32)

        def zb(v, _):
            acc[pl.ds(v * 16, 16)] = z
            return 0

        lax.fori_loop(0, SC_CW // 16, zb, 0)

        for i in range(SC_NCH):
            slot = i % SC_RING
            mk(i, slot).wait()

            def vgb(vg, _):
                base = vg * 64
                a0 = acc[pl.ds(base, 16)]
                a1 = acc[pl.ds(base + 16, 16)]
                a2 = acc[pl.ds(base + 32, 16)]
                a3 = acc[pl.ds(base + 48, 16)]

                def kb(kk, carry):
                    b0, b1, b2v, b3 = carry
                    h = hbuf[i * SC_KT + kk]
                    b0 = b0 + h * wbuf[slot, kk, pl.ds(base, 16)]
                    b1 = b1 + h * wbuf[slot, kk, pl.ds(base + 16, 16)]
                    b2v = b2v + h * wbuf[slot, kk, pl.ds(base + 32, 16)]
                    b3 = b3 + h * wbuf[slot, kk, pl.ds(base + 48, 16)]
                    return (b0, b1, b2v, b3)

                a0, a1, a2, a3 = lax.fori_loop(0, SC_KT, kb, (a0, a1, a2, a3))
                acc[pl.ds(base, 16)] = a0
                acc[pl.ds(base + 16, 16)] = a1
                acc[pl.ds(base + 32, 16)] = a2
                acc[pl.ds(base + 48, 16)] = a3
                return 0

            lax.fori_loop(0, SC_CW // 64, vgb, 0)
            nxt = i + SC_RING - 1
            if nxt < SC_NCH:
                mk(nxt, nxt % SC_RING).start()

        pltpu.sync_copy(acc, out_hbm.at[pl.ds(wid * SC_CW, SC_CW)])

    return k(W2, hid_b)


def _hid_body(e_ref, w1_ref, b1_ref, o_ref):
    o_ref[...] = jnp.maximum(
        jnp.dot(e_ref[...], w1_ref[...], preferred_element_type=jnp.float32)
        + b1_ref[...],
        0.0,
    )


def _make_out_body(coff, cw):
    def _out_body(hid_ref, w2_hbm, o_ref, bufs, sems):
        def mk(q):
            return pltpu.make_async_copy(
                w2_hbm.at[:, q, :, pl.ds(coff, cw)],
                bufs.at[q % NBUF],
                sems.at[q % NBUF],
            )

        for s in range(NBUF - 1):
            mk(s).start()
        for q in range(NQ):
            mk(q).wait()
            w = bufs[q % NBUF].reshape(NST * 8, cw)
            t = jnp.dot(hid_ref[q], w, preferred_element_type=jnp.float32)
            if q == 0:
                o_ref[...] = t
            else:
                o_ref[...] = o_ref[...] + t
            nxt = q + NBUF - 1
            if nxt < NQ:
                mk(nxt).start()

    return _out_body


def _tc_stream(hid_p, W2v, coff, cw):
    return pl.pallas_call(
        _make_out_body(coff, cw),
        in_specs=[
            pl.BlockSpec((NQ, 1, NST * 8), lambda: (0, 0, 0)),
            pl.BlockSpec(memory_space=pl.ANY),
        ],
        out_specs=pl.BlockSpec((1, cw), lambda: (0, 0)),
        out_shape=jax.ShapeDtypeStruct((1, cw), jnp.float32),
        scratch_shapes=[
            pltpu.VMEM((NBUF, NST, 8, cw), jnp.float32),
            pltpu.SemaphoreType.DMA((NBUF,)),
        ],
    )(hid_p, W2v)


def _fin_body(tc_ref, sc_ref, hid_ref, wt_ref, b2_ref, o_ref):
    o_ref[:, : C_TC] = tc_ref[...] + b2_ref[:, : C_TC]
    o_ref[:, C_TC : C_TC + SC_COLS] = (
        sc_ref[...] + b2_ref[:, C_TC : C_TC + SC_COLS]
    )
    t = jnp.dot(hid_ref[...], wt_ref[...], preferred_element_type=jnp.float32)
    o_ref[:, C_TC + SC_COLS :] = t + b2_ref[:, C_TC + SC_COLS :]
    full = o_ref[...]
    m = jnp.max(full)
    s = jnp.sum(jnp.exp(full - m))
    o_ref[...] = full - (m + jnp.log(s))


def kernel(inputs, table, W1, b1, W2, b2):
    idx = inputs.astype(jnp.int32)
    emb = _sc_gather(table, idx)
    emb_flat = emb.reshape(1, NIDX * EMBD)

    hid = pl.pallas_call(
        _hid_body,
        out_shape=jax.ShapeDtypeStruct((1, HID), jnp.float32),
    )(emb_flat, W1, b1.reshape(1, HID))

    hid_p = jnp.transpose(hid.reshape(NST, NQ, 8), (1, 0, 2)).reshape(NQ, 1, NST * 8)
    W2v = W2.reshape(NST, NQ, 8, VOCAB)

    C_LO = 33280
    tc_lo = _tc_stream(hid_p, W2v, 0, C_LO)
    sc_part = _sc_matvec(W2, jnp.broadcast_to(hid.reshape(HID, 1), (HID, 16)))
    tc_hi = _tc_stream(hid_p, W2v, C_LO, C_TC - C_LO)

    tc_part = jnp.concatenate([tc_lo, tc_hi], axis=1)
    w2_tail = lax.slice(W2, (0, VOCAB - TAIL), (HID, VOCAB))

    log_probs = pl.pallas_call(
        _fin_body,
        out_shape=jax.ShapeDtypeStruct((1, VOCAB), jnp.float32),
    )(tc_part, sc_part.reshape(1, SC_COLS), hid, w2_tail, b2.reshape(1, VOCAB))

    return log_probs


# speedup vs baseline: 1.2160x; 1.2160x over previous
import functools

import jax
import jax.numpy as jnp
from jax import lax
from jax.experimental import pallas as pl
from jax.experimental.pallas import tpu as pltpu
from jax.experimental.pallas import tpu_sc as plsc

VOCAB = 100000
EMBD = 128
CTX = 10
HID = 512
NIDX = 2 * CTX

NQ = 16
NST = 4
NBUF = 3


def _sc_gather(table, idx):
    mesh = plsc.VectorSubcoreMesh(core_axis_name="c", subcore_axis_name="s")

    @functools.partial(
        pl.kernel,
        mesh=mesh,
        out_type=jax.ShapeDtypeStruct((NIDX, EMBD), jnp.float32),
        scratch_types=[
            pltpu.VMEM((NIDX,), jnp.int32),
            pltpu.VMEM((NIDX, EMBD), jnp.float32),
            pltpu.SemaphoreType.DMA,
        ],
    )
    def gather_k(table_hbm, idx_hbm, out_hbm, idx_v, rows_v, sem):
        wid = lax.axis_index("s") * 2 + lax.axis_index("c")

        @pl.when(wid == 0)
        def _():
            pltpu.sync_copy(idx_hbm, idx_v)
            pltpu.async_copy(table_hbm.at[idx_v], rows_v, sem).wait()
            pltpu.sync_copy(rows_v, out_hbm)

    return gather_k(table, idx)


def _hid_body(e_ref, w1_ref, b1_ref, o_ref):
    o_ref[...] = jnp.maximum(
        jnp.dot(e_ref[...], w1_ref[...], preferred_element_type=jnp.float32)
        + b1_ref[...],
        0.0,
    )


def _out_body(hid_ref, b2_ref, w2_hbm, o_ref, bufs, sems):
    def mk(q):
        return pltpu.make_async_copy(
            w2_hbm.at[:, q, :, :], bufs.at[q % NBUF], sems.at[q % NBUF]
        )

    for s in range(NBUF - 1):
        mk(s).start()
    for q in range(NQ):
        mk(q).wait()
        w = bufs[q % NBUF].reshape(NST * 8, VOCAB)
        t = jnp.dot(hid_ref[q], w, preferred_element_type=jnp.float32)
        if q == 0:
            o_ref[...] = t + b2_ref[...]
        else:
            o_ref[...] = o_ref[...] + t
        nxt = q + NBUF - 1
        if nxt < NQ:
            mk(nxt).start()

    full = o_ref[...]
    m = jnp.max(full)
    s = jnp.sum(jnp.exp(full - m))
    o_ref[...] = full - (m + jnp.log(s))


def kernel(inputs, table, W1, b1, W2, b2):
    idx = inputs.astype(jnp.int32)
    emb = _sc_gather(table, idx)
    emb_flat = emb.reshape(1, NIDX * EMBD)

    hid = pl.pallas_call(
        _hid_body,
        out_shape=jax.ShapeDtypeStruct((1, HID), jnp.float32),
    )(emb_flat, W1, b1.reshape(1, HID))

    hid_p = jnp.transpose(hid.reshape(NST, NQ, 8), (1, 0, 2)).reshape(NQ, 1, NST * 8)

    log_probs = pl.pallas_call(
        _out_body,
        in_specs=[
            pl.BlockSpec((NQ, 1, NST * 8), lambda: (0, 0, 0)),
            pl.BlockSpec((1, VOCAB), lambda: (0, 0)),
            pl.BlockSpec(memory_space=pl.ANY),
        ],
        out_specs=pl.BlockSpec((1, VOCAB), lambda: (0, 0)),
        out_shape=jax.ShapeDtypeStruct((1, VOCAB), jnp.float32),
        scratch_shapes=[
            pltpu.VMEM((NBUF, NST, 8, VOCAB), jnp.float32),
            pltpu.SemaphoreType.DMA((NBUF,)),
        ],
    )(hid_p, b2.reshape(1, VOCAB), W2.reshape(NST, NQ, 8, VOCAB))

    return log_probs
